# baseline (speedup 1.0000x reference)
import functools

import jax
import jax.numpy as jnp
from jax import lax
from jax.experimental import pallas as pl
from jax.experimental.pallas import tpu as pltpu
from jax.experimental.pallas import tpu_sc as plsc

HIDDEN = 64
K = 20
NC, NS, L = 2, 16, 16
NW = NC * NS
BATCH, SEQ = 1024, 50
ROWS_PER_W = BATCH // NW
IDX_PER_W = ROWS_PER_W * SEQ * K
CHUNK_B = 10
N_CHUNKS = ROWS_PER_W * SEQ // CHUNK_B
ROWS_PER_CHUNK = CHUNK_B * K
GATHER_SPLITS = ((0, 128), (128, 72))
NVREG = HIDDEN // L


def _body(idx_hbm, table_hbm, out_hbm, idx_v, rows_v, out_v,
          gsem0, gsem1, osem0, osem1):
    wid = lax.axis_index("s") * NC + lax.axis_index("c")

    def fire_gather(g, slot, sem):
        for off, n in GATHER_SPLITS:
            pltpu.async_copy(
                table_hbm.at[idx_v.at[pl.ds(g * ROWS_PER_CHUNK + off, n)]],
                rows_v.at[slot, pl.ds(off, n)],
                sem)

    def wait_gather(slot, sem):
        for off, n in GATHER_SPLITS:
            pltpu.make_async_copy(
                table_hbm.at[idx_v.at[pl.ds(off, n)]],
                rows_v.at[slot, pl.ds(off, n)],
                sem).wait()

    def compute_chunk(g, slot):
        @pl.loop(0, CHUNK_B, unroll=5)
        def basket(c):
            base = c * K
            for j in range(NVREG):
                vs = [rows_v[slot, base + k, pl.ds(j * L, L)] +
                      rows_v[slot, base + k + 1, pl.ds(j * L, L)]
                      for k in range(0, K, 2)]
                while len(vs) > 1:
                    nxt_vs = [vs[i] + vs[i + 1] for i in range(0, len(vs) - 1, 2)]
                    if len(vs) % 2:
                        nxt_vs.append(vs[-1])
                    vs = nxt_vs
                out_v[slot, c, pl.ds(j * L, L)] = vs[0] * jnp.float32(1.0 / K)
        pltpu.async_copy(
            out_v.at[slot],
            out_hbm.at[wid * ROWS_PER_W + g // 5,
                       pl.ds((g % 5) * CHUNK_B, CHUNK_B), :],
            osems[slot],
        )

    def wait_out(slot):
        pltpu.make_async_copy(
            out_v.at[slot],
            out_hbm.at[wid * ROWS_PER_W, pl.ds(0, CHUNK_B), :],
            osems[slot],
        ).wait()

    gsems = (gsem0, gsem1)
    osems = (osem0, osem1)

    pltpu.sync_copy(idx_hbm.at[pl.ds(wid * IDX_PER_W, IDX_PER_W)], idx_v)
    fire_gather(0, 0, gsem0)

    @pl.loop(0, N_CHUNKS, step=2)
    def _chunks(g0):
        for b in range(2):
            g = g0 + b
            nxt = 1 - b
            if b == 0:
                fire_gather(g + 1, nxt, gsems[nxt])
            else:
                @pl.when(g0 < N_CHUNKS - 2)
                def _():
                    fire_gather(g + 1, nxt, gsems[nxt])
            wait_gather(b, gsems[b])
            @pl.when(g >= 2)
            def _():
                wait_out(b)
            compute_chunk(g, b)

    wait_out(0)
    wait_out(1)


@jax.jit
def _pooled(idx, table):
    mesh = plsc.VectorSubcoreMesh(
        core_axis_name="c", subcore_axis_name="s",
        num_cores=NC, num_subcores=NS,
    )
    run = functools.partial(
        pl.kernel,
        out_type=jax.ShapeDtypeStruct((BATCH, SEQ, HIDDEN), jnp.float32),
        mesh=mesh,
        compiler_params=pltpu.CompilerParams(use_tc_tiling_on_sc=False),
        scratch_types=[
            pltpu.VMEM((IDX_PER_W,), jnp.int32),
            pltpu.VMEM((2, ROWS_PER_CHUNK, HIDDEN), jnp.float32),
            pltpu.VMEM((2, CHUNK_B, HIDDEN), jnp.float32),
            pltpu.SemaphoreType.DMA,
            pltpu.SemaphoreType.DMA,
            pltpu.SemaphoreType.DMA,
            pltpu.SemaphoreType.DMA,
        ],
    )(_body)
    return run(idx, table)


def kernel(batch_basket, table):
    return _pooled(batch_basket.reshape(-1), table)
